# baseline (speedup 1.0000x reference)
import jax
import jax.numpy as jnp
from jax import lax
from jax.experimental import pallas as pl
from jax.experimental.pallas import tpu as pltpu
from jax.experimental.pallas import tpu_sc as plsc

B = 4096
L = 200
DIM = 64
N = B * L

NUM_CORES = 2
NUM_SUBCORES = 16
NW = NUM_CORES * NUM_SUBCORES
PER_W = N // NW
CHUNK = 512
NCHUNK = PER_W // CHUNK
assert NCHUNK % 2 == 0


def _emb_body(idx_hbm, table_hbm, out_hbm,
              idx_v0, idx_v1, rows_v0, rows_v1,
              gsem0, gsem1, osem0, osem1):
    wid = lax.axis_index("s") * NUM_CORES + lax.axis_index("c")
    base = wid * PER_W

    idx_v = (idx_v0, idx_v1)
    rows_v = (rows_v0, rows_v1)
    gsem = (gsem0, gsem1)
    osem = (osem0, osem1)

    def start_gather(g, b):
        pltpu.sync_copy(idx_hbm.at[pl.ds(base + g * CHUNK, CHUNK)], idx_v[b])
        pltpu.make_async_copy(table_hbm.at[idx_v[b]], rows_v[b],
                              gsem[b]).start()

    def wait_gather(b):
        pltpu.make_async_copy(table_hbm.at[idx_v[b]], rows_v[b],
                              gsem[b]).wait()

    def start_write(g, b):
        pltpu.make_async_copy(rows_v[b],
                              out_hbm.at[pl.ds(base + g * CHUNK, CHUNK)],
                              osem[b]).start()

    def wait_write(g, b):
        pltpu.make_async_copy(rows_v[b],
                              out_hbm.at[pl.ds(base + g * CHUNK, CHUNK)],
                              osem[b]).wait()

    start_gather(0, 0)

    def body(gg, _):
        for b in range(2):
            g = gg * 2 + b
            nb = 1 - b

            @pl.when(g + 1 < NCHUNK)
            def _():
                @pl.when(g >= 1)
                def _():
                    wait_write(g - 1, nb)
                start_gather(g + 1, nb)

            wait_gather(b)
            start_write(g, b)
        return 0

    lax.fori_loop(0, NCHUNK // 2, body, 0)
    wait_write(NCHUNK - 2, 0)
    wait_write(NCHUNK - 1, 1)


def kernel(x, table):
    xf = x.reshape(N)
    mesh = plsc.VectorSubcoreMesh(core_axis_name="c", subcore_axis_name="s",
                                  num_cores=NUM_CORES,
                                  num_subcores=NUM_SUBCORES)
    out = pl.kernel(
        _emb_body,
        out_type=jax.ShapeDtypeStruct((N, DIM), jnp.float32),
        mesh=mesh,
        scratch_types=[
            pltpu.VMEM((CHUNK,), jnp.int32),
            pltpu.VMEM((CHUNK,), jnp.int32),
            pltpu.VMEM((CHUNK, DIM), jnp.float32),
            pltpu.VMEM((CHUNK, DIM), jnp.float32),
            pltpu.SemaphoreType.DMA,
            pltpu.SemaphoreType.DMA,
            pltpu.SemaphoreType.DMA,
            pltpu.SemaphoreType.DMA,
        ],
        compiler_params=pltpu.CompilerParams(use_tc_tiling_on_sc=False),
    )(xf, table)
    return out.reshape(B, L, DIM)


# speedup vs baseline: 1.2267x; 1.2267x over previous
import jax
import jax.numpy as jnp
from jax import lax
from jax.experimental import pallas as pl
from jax.experimental.pallas import tpu as pltpu
from jax.experimental.pallas import tpu_sc as plsc

B = 4096
L = 200
DIM = 64
DPAD = 128
N = B * L

NUM_CORES = 2
NUM_SUBCORES = 16
NW = NUM_CORES * NUM_SUBCORES
PER_W = N // NW
CHUNK = 256
NCHUNK = PER_W // CHUNK
assert NCHUNK % 2 == 0


def _emb_body(idx_hbm, table_hbm, out_hbm,
              idx_v0, idx_v1, rows_v0, rows_v1,
              gsem0, gsem1, osem0, osem1):
    wid = lax.axis_index("s") * NUM_CORES + lax.axis_index("c")
    base = wid * PER_W

    idx_v = (idx_v0, idx_v1)
    rows_v = (rows_v0, rows_v1)
    gsem = (gsem0, gsem1)
    osem = (osem0, osem1)

    def start_gather(g, b):
        pltpu.sync_copy(idx_hbm.at[pl.ds(base + g * CHUNK, CHUNK)], idx_v[b])
        pltpu.make_async_copy(table_hbm.at[idx_v[b]], rows_v[b],
                              gsem[b]).start()

    def wait_gather(b):
        pltpu.make_async_copy(table_hbm.at[idx_v[b]], rows_v[b],
                              gsem[b]).wait()

    def start_write(g, b):
        pltpu.make_async_copy(rows_v[b],
                              out_hbm.at[pl.ds(base + g * CHUNK, CHUNK)],
                              osem[b]).start()

    def wait_write(g, b):
        pltpu.make_async_copy(rows_v[b],
                              out_hbm.at[pl.ds(base + g * CHUNK, CHUNK)],
                              osem[b]).wait()

    start_gather(0, 0)

    def body(gg, _):
        for b in range(2):
            g = gg * 2 + b
            nb = 1 - b

            @pl.when(g + 1 < NCHUNK)
            def _():
                @pl.when(g >= 1)
                def _():
                    wait_write(g - 1, nb)
                start_gather(g + 1, nb)

            wait_gather(b)
            start_write(g, b)
        return 0

    lax.fori_loop(0, NCHUNK // 2, body, 0)
    wait_write(NCHUNK - 2, 0)
    wait_write(NCHUNK - 1, 1)


def kernel(x, table):
    xf = x.reshape(N)
    tbl_pad = jnp.pad(table, ((0, 0), (0, DPAD - DIM)))
    mesh = plsc.VectorSubcoreMesh(core_axis_name="c", subcore_axis_name="s",
                                  num_cores=NUM_CORES,
                                  num_subcores=NUM_SUBCORES)
    out_pad = pl.kernel(
        _emb_body,
        out_type=jax.ShapeDtypeStruct((N, DPAD), jnp.float32),
        mesh=mesh,
        scratch_types=[
            pltpu.VMEM((CHUNK,), jnp.int32),
            pltpu.VMEM((CHUNK,), jnp.int32),
            pltpu.VMEM((CHUNK, DPAD), jnp.float32),
            pltpu.VMEM((CHUNK, DPAD), jnp.float32),
            pltpu.SemaphoreType.DMA,
            pltpu.SemaphoreType.DMA,
            pltpu.SemaphoreType.DMA,
            pltpu.SemaphoreType.DMA,
        ],
        compiler_params=pltpu.CompilerParams(use_tc_tiling_on_sc=False),
    )(xf, tbl_pad)
    return out_pad[:, :DIM].reshape(B, L, DIM)
